# baseline (speedup 1.0000x reference)
import functools

import jax
import jax.numpy as jnp
from jax import lax
from jax.experimental import pallas as pl
from jax.experimental.pallas import tpu as pltpu
from jax.experimental.pallas import tpu_sc as plsc

T = 16384
D = 2048
E = 8
K = 2

NC, NS, L = 2, 16, 16
NW = NC * NS

MMBLK = 512
RING = 4

C0 = 4096
C1 = T - C0


def _mm_body(row0, nstep, h_hbm, w_ref, o_ref, bufs, sems):
    def copy(i):
        return pltpu.make_async_copy(
            h_hbm.at[pl.ds(row0 + i * MMBLK, MMBLK), :],
            bufs.at[i % RING],
            sems.at[i % RING],
        )

    for i in range(min(RING - 1, nstep)):
        copy(i).start()
    for i in range(nstep):
        if i + RING - 1 < nstep:
            copy(i + RING - 1).start()
        copy(i).wait()
        o_ref[:, pl.ds(i * MMBLK, MMBLK)] = lax.dot_general(
            w_ref[...], bufs[i % RING],
            (((1,), (1,)), ((), ())),
            preferred_element_type=jnp.float32,
        )


def _logits_tc(hidden_states, W_gate, row0, tokens):
    return pl.pallas_call(
        functools.partial(_mm_body, row0, tokens // MMBLK),
        in_specs=[
            pl.BlockSpec(memory_space=pltpu.HBM),
            pl.BlockSpec(memory_space=pltpu.VMEM),
        ],
        out_specs=pl.BlockSpec(memory_space=pltpu.VMEM),
        out_shape=jax.ShapeDtypeStruct((E, tokens), jnp.float32),
        scratch_shapes=[
            pltpu.VMEM((RING, MMBLK, D), jnp.float32),
            pltpu.SemaphoreType.DMA((RING,)),
        ],
    )(hidden_states, W_gate)


def _route_body(chunk, logits_hbm, vals_hbm, idx_hbm, lv, wv, iv):
    wid = lax.axis_index("s") * NC + lax.axis_index("c")
    base = wid * chunk
    pltpu.sync_copy(logits_hbm.at[:, pl.ds(base, chunk)], lv)

    def group(g, carry):
        sl = pl.ds(g * L, L)
        m1 = lv[0, sl]
        i1 = jnp.zeros((L,), jnp.int32)
        m2 = jnp.full((L,), -jnp.inf, jnp.float32)
        i2 = jnp.zeros((L,), jnp.int32)
        for e in range(1, E):
            l = lv[e, sl]
            ev = jnp.full((L,), e, jnp.int32)
            gt1 = l > m1
            gt2 = l > m2
            m2 = jnp.where(gt1, m1, jnp.where(gt2, l, m2))
            i2 = jnp.where(gt1, i1, jnp.where(gt2, ev, i2))
            m1 = jnp.where(gt1, l, m1)
            i1 = jnp.where(gt1, ev, i1)
        w1 = 1.0 / (1.0 + jnp.exp(m2 - m1))
        wv[0, sl] = w1
        wv[1, sl] = 1.0 - w1
        iv[0, sl] = i1
        iv[1, sl] = i2
        return carry

    lax.fori_loop(0, chunk // L, group, 0)
    pltpu.sync_copy(wv, vals_hbm.at[:, pl.ds(base, chunk)])
    pltpu.sync_copy(iv, idx_hbm.at[:, pl.ds(base, chunk)])


@functools.lru_cache(maxsize=None)
def _make_route_sc(tokens):
    chunk = tokens // NW
    return pl.kernel(
        functools.partial(_route_body, chunk),
        mesh=plsc.VectorSubcoreMesh(
            core_axis_name="c", subcore_axis_name="s",
            num_cores=NC, num_subcores=NS,
        ),
        out_type=[
            jax.ShapeDtypeStruct((K, tokens), jnp.float32),
            jax.ShapeDtypeStruct((K, tokens), jnp.int32),
        ],
        scratch_types=[
            pltpu.VMEM((E, chunk), jnp.float32),
            pltpu.VMEM((K, chunk), jnp.float32),
            pltpu.VMEM((K, chunk), jnp.int32),
        ],
    )


@jax.jit
def kernel(hidden_states, W_gate):
    l0 = _logits_tc(hidden_states, W_gate, 0, C0)
    l1 = _logits_tc(hidden_states, W_gate, C0, C1)
    v0, i0 = _make_route_sc(C0)(l0)
    v1, i1 = _make_route_sc(C1)(l1)
    vals = jnp.concatenate([v0, v1], axis=1).T
    idx = jnp.concatenate([i0, i1], axis=1).T
    return vals, idx


# speedup vs baseline: 1.1049x; 1.1049x over previous
import functools

import jax
import jax.numpy as jnp
from jax import lax
from jax.experimental import pallas as pl
from jax.experimental.pallas import tpu as pltpu
from jax.experimental.pallas import tpu_sc as plsc

T = 16384
D = 2048
E = 8
K = 2

NC, NS, L = 2, 16, 16
NW = NC * NS
CHUNK = T // NW
NG = CHUNK // L

MMBLK = 512
HALF = MMBLK // 2
NSTEP = T // MMBLK
RING = 4


def _mm_body(h_hbm, w_ref, o_ref, bufs, sems_a, sems_b):
    def copies(i):
        s = i % RING
        return (
            pltpu.make_async_copy(
                h_hbm.at[pl.ds(i * MMBLK, HALF), :],
                bufs.at[s, pl.ds(0, HALF)],
                sems_a.at[s],
            ),
            pltpu.make_async_copy(
                h_hbm.at[pl.ds(i * MMBLK + HALF, HALF), :],
                bufs.at[s, pl.ds(HALF, HALF)],
                sems_b.at[s],
            ),
        )

    def start(i):
        ca, cb = copies(i)
        ca.start()
        cb.start()

    def wait(i):
        ca, cb = copies(i)
        ca.wait()
        cb.wait()

    for i in range(RING - 1):
        start(i)
    for i in range(NSTEP):
        if i + RING - 1 < NSTEP:
            start(i + RING - 1)
        wait(i)
        o_ref[:, pl.ds(i * MMBLK, MMBLK)] = lax.dot_general(
            w_ref[...], bufs[i % RING],
            (((1,), (1,)), ((), ())),
            preferred_element_type=jnp.float32,
        )


def _logits_tc(hidden_states, W_gate):
    return pl.pallas_call(
        _mm_body,
        in_specs=[
            pl.BlockSpec(memory_space=pltpu.HBM),
            pl.BlockSpec(memory_space=pltpu.VMEM),
        ],
        out_specs=pl.BlockSpec(memory_space=pltpu.VMEM),
        out_shape=jax.ShapeDtypeStruct((E, T), jnp.float32),
        scratch_shapes=[
            pltpu.VMEM((RING, MMBLK, D), jnp.float32),
            pltpu.SemaphoreType.DMA((RING,)),
            pltpu.SemaphoreType.DMA((RING,)),
        ],
    )(hidden_states, W_gate)


def _route_body(logits_hbm, vals_hbm, idx_hbm, lv, wv, iv):
    wid = lax.axis_index("s") * NC + lax.axis_index("c")
    base = wid * CHUNK
    pltpu.sync_copy(logits_hbm.at[:, pl.ds(base, CHUNK)], lv)

    def group(g, carry):
        sl = pl.ds(g * L, L)
        m1 = lv[0, sl]
        i1 = jnp.zeros((L,), jnp.int32)
        m2 = jnp.full((L,), -jnp.inf, jnp.float32)
        i2 = jnp.zeros((L,), jnp.int32)
        for e in range(1, E):
            l = lv[e, sl]
            ev = jnp.full((L,), e, jnp.int32)
            gt1 = l > m1
            gt2 = l > m2
            m2 = jnp.where(gt1, m1, jnp.where(gt2, l, m2))
            i2 = jnp.where(gt1, i1, jnp.where(gt2, ev, i2))
            m1 = jnp.where(gt1, l, m1)
            i1 = jnp.where(gt1, ev, i1)
        w1 = 1.0 / (1.0 + jnp.exp(m2 - m1))
        wv[0, sl] = w1
        wv[1, sl] = 1.0 - w1
        iv[0, sl] = i1
        iv[1, sl] = i2
        return carry

    lax.fori_loop(0, NG, group, 0)
    pltpu.sync_copy(wv, vals_hbm.at[:, pl.ds(base, CHUNK)])
    pltpu.sync_copy(iv, idx_hbm.at[:, pl.ds(base, CHUNK)])


@functools.lru_cache(maxsize=None)
def _make_route_sc():
    return pl.kernel(
        _route_body,
        mesh=plsc.VectorSubcoreMesh(
            core_axis_name="c", subcore_axis_name="s",
            num_cores=NC, num_subcores=NS,
        ),
        out_type=[
            jax.ShapeDtypeStruct((K, T), jnp.float32),
            jax.ShapeDtypeStruct((K, T), jnp.int32),
        ],
        scratch_types=[
            pltpu.VMEM((E, CHUNK), jnp.float32),
            pltpu.VMEM((K, CHUNK), jnp.float32),
            pltpu.VMEM((K, CHUNK), jnp.int32),
        ],
    )


@jax.jit
def kernel(hidden_states, W_gate):
    logits = _logits_tc(hidden_states, W_gate)
    vals, idx = _make_route_sc()(logits)
    return vals.T, idx.T


# speedup vs baseline: 1.1087x; 1.0034x over previous
import functools

import jax
import jax.numpy as jnp
from jax import lax
from jax.experimental import pallas as pl
from jax.experimental.pallas import tpu as pltpu
from jax.experimental.pallas import tpu_sc as plsc

T = 16384
D = 2048
E = 8
K = 2

NC, NS, L = 1, 16, 16
NW = NC * NS
CHUNK = T // NW
NG = CHUNK // L

MMBLK = 512
HALF = MMBLK // 2
NSTEP = T // MMBLK
RING = 4


def _mm_body(h_hbm, w_ref, o_ref, bufs, sems_a, sems_b):
    def copies(i):
        s = i % RING
        return (
            pltpu.make_async_copy(
                h_hbm.at[pl.ds(i * MMBLK, HALF), :],
                bufs.at[s, pl.ds(0, HALF)],
                sems_a.at[s],
            ),
            pltpu.make_async_copy(
                h_hbm.at[pl.ds(i * MMBLK + HALF, HALF), :],
                bufs.at[s, pl.ds(HALF, HALF)],
                sems_b.at[s],
            ),
        )

    def start(i):
        ca, cb = copies(i)
        ca.start()
        cb.start()

    def wait(i):
        ca, cb = copies(i)
        ca.wait()
        cb.wait()

    for i in range(RING - 1):
        start(i)
    for i in range(NSTEP):
        if i + RING - 1 < NSTEP:
            start(i + RING - 1)
        wait(i)
        o_ref[:, pl.ds(i * MMBLK, MMBLK)] = lax.dot_general(
            w_ref[...], bufs[i % RING],
            (((1,), (1,)), ((), ())),
            preferred_element_type=jnp.float32,
        )


def _logits_tc(hidden_states, W_gate):
    return pl.pallas_call(
        _mm_body,
        in_specs=[
            pl.BlockSpec(memory_space=pltpu.HBM),
            pl.BlockSpec(memory_space=pltpu.VMEM),
        ],
        out_specs=pl.BlockSpec(memory_space=pltpu.VMEM),
        out_shape=jax.ShapeDtypeStruct((E, T), jnp.float32),
        scratch_shapes=[
            pltpu.VMEM((RING, MMBLK, D), jnp.float32),
            pltpu.SemaphoreType.DMA((RING,)),
            pltpu.SemaphoreType.DMA((RING,)),
        ],
    )(hidden_states, W_gate)


def _route_body(logits_hbm, vals_hbm, idx_hbm, lv, wv, iv):
    wid = lax.axis_index("s") * NC + lax.axis_index("c")
    base = wid * CHUNK
    pltpu.sync_copy(logits_hbm.at[:, pl.ds(base, CHUNK)], lv)

    def group(g, carry):
        sl = pl.ds(g * L, L)
        m1 = lv[0, sl]
        i1 = jnp.zeros((L,), jnp.int32)
        m2 = jnp.full((L,), -jnp.inf, jnp.float32)
        i2 = jnp.zeros((L,), jnp.int32)
        for e in range(1, E):
            l = lv[e, sl]
            ev = jnp.full((L,), e, jnp.int32)
            gt1 = l > m1
            gt2 = l > m2
            m2 = jnp.where(gt1, m1, jnp.where(gt2, l, m2))
            i2 = jnp.where(gt1, i1, jnp.where(gt2, ev, i2))
            m1 = jnp.where(gt1, l, m1)
            i1 = jnp.where(gt1, ev, i1)
        w1 = 1.0 / (1.0 + jnp.exp(m2 - m1))
        wv[0, sl] = w1
        wv[1, sl] = 1.0 - w1
        iv[0, sl] = i1
        iv[1, sl] = i2
        return carry

    lax.fori_loop(0, NG, group, 0)
    pltpu.sync_copy(wv, vals_hbm.at[:, pl.ds(base, CHUNK)])
    pltpu.sync_copy(iv, idx_hbm.at[:, pl.ds(base, CHUNK)])


@functools.lru_cache(maxsize=None)
def _make_route_sc():
    return pl.kernel(
        _route_body,
        mesh=plsc.VectorSubcoreMesh(
            core_axis_name="c", subcore_axis_name="s",
            num_cores=NC, num_subcores=NS,
        ),
        out_type=[
            jax.ShapeDtypeStruct((K, T), jnp.float32),
            jax.ShapeDtypeStruct((K, T), jnp.int32),
        ],
        scratch_types=[
            pltpu.VMEM((E, CHUNK), jnp.float32),
            pltpu.VMEM((K, CHUNK), jnp.float32),
            pltpu.VMEM((K, CHUNK), jnp.int32),
        ],
    )


@jax.jit
def kernel(hidden_states, W_gate):
    logits = _logits_tc(hidden_states, W_gate)
    vals, idx = _make_route_sc()(logits)
    return vals.T, idx.T
